# baseline (speedup 1.0000x reference)
import functools

import jax
import jax.numpy as jnp
from jax import lax
from jax.experimental import pallas as pl
from jax.experimental.pallas import tpu as pltpu
from jax.experimental.pallas import tpu_sc as plsc

N = 10000
E = 320000
D = 128
H = 128
C = 40
C_PAD = 48

NC = 2
NS = 16
NW = NC * NS

NP = 10240
ROWS_PER_SUB = NP // NS

CK = 128
EW = 10240
CH = EW // CK
EP = EW * NW

EW0 = 10240
EW1 = 10240
EP_U = NS * (EW0 + EW1)

_mesh = plsc.VectorSubcoreMesh(core_axis_name="c", subcore_axis_name="s")
_sc_params = pltpu.CompilerParams(
    needs_layout_passes=False, use_tc_tiling_on_sc=False
)


def _full16(v):
    return jnp.full((16,), v, dtype=jnp.int32)


def _deg_body(colm, wm, zeros1, out, col_v, w_v, sem, degsh):
    c = lax.axis_index("c")
    s = lax.axis_index("s")
    wid = s * NC + c
    pltpu.sync_copy(colm.at[wid], col_v)
    pltpu.sync_copy(wm.at[wid], w_v)
    pltpu.sync_copy(zeros1, degsh.at[pl.ds(s * ROWS_PER_SUB, ROWS_PER_SUB)])
    plsc.subcore_barrier()

    def chunk(ci, carry):
        pltpu.sync_copy(w_v.at[ci], degsh.at[col_v.at[ci]], add=True)
        return carry

    lax.fori_loop(0, CH, chunk, 0)
    plsc.subcore_barrier()
    sl = pl.ds(s * ROWS_PER_SUB, ROWS_PER_SUB)
    pltpu.sync_copy(degsh.at[sl], out.at[c, sl])


_deg_kernel = functools.partial(
    pl.kernel,
    out_type=jax.ShapeDtypeStruct((NC, NP), jnp.float32),
    mesh=_mesh,
    scratch_types=[
        pltpu.VMEM((CH, CK), jnp.int32),
        pltpu.VMEM((CH, CK), jnp.float32),
        pltpu.SemaphoreType.DMA,
        pltpu.VMEM_SHARED((NP,), jnp.float32),
    ],
    compiler_params=_sc_params,
)(_deg_body)


def _edge_body(dw, dwb, ck, g, rowm, colm, wmf, zerosD, out, row_v, col_v,
               w_v, buf0, buf1, fbuf, sem0, sem1, accsh):
    ch0 = EW0 // ck
    ch1 = EW1 // ck
    c = lax.axis_index("c")
    s = lax.axis_index("s")
    wid = c * NS + s
    chv = ch0 if ch0 == ch1 else jnp.where(c == 0, ch0, ch1)
    pltpu.sync_copy(rowm.at[wid], row_v)
    pltpu.sync_copy(colm.at[wid], col_v)
    pltpu.sync_copy(wmf.at[wid], w_v)
    pltpu.sync_copy(zerosD, accsh.at[pl.ds(s * ROWS_PER_SUB, ROWS_PER_SUB)])
    plsc.subcore_barrier()

    mneg = jnp.full((16,), -65536, jnp.int32)

    def scale_rows(buf, ci):
        def block(b, bc):
            rb = b * 16
            for k in range(16):
                wspl = plsc.load_gather(w_v, [_full16(ci * ck + rb + k)])
                r = rb + k
                for j in range(dwb // 32):
                    v = plsc.bitcast(buf[r, pl.ds(j * 32, 32)], jnp.int32)
                    a = plsc.bitcast(v << 16, jnp.float32)
                    fbuf[r, pl.ds(j * 32, 16)] = a * wspl
                    if j * 32 + 16 < dw:
                        bb = plsc.bitcast(v & mneg, jnp.float32)
                        fbuf[r, pl.ds(j * 32 + 16, 16)] = bb * wspl
            return bc

        lax.fori_loop(0, ck // 16, block, 0)

    def start_gather(ci, buf, sem):
        pltpu.async_copy(g.at[row_v.at[ci]], buf, sem)

    def wait_gather(ci, buf, sem):
        pltpu.make_async_copy(g.at[row_v.at[ci]], buf, sem).wait()

    start_gather(0, buf0, sem0)
    start_gather(1, buf1, sem1)

    def body(k, carry):
        c0 = 2 * k
        c1 = c0 + 1
        wait_gather(c0, buf0, sem0)
        scale_rows(buf0, c0)

        @pl.when(c0 + 2 < chv)
        def _():
            start_gather(c0 + 2, buf0, sem0)

        pltpu.sync_copy(fbuf, accsh.at[col_v.at[c0]], add=True)

        wait_gather(c1, buf1, sem1)
        scale_rows(buf1, c1)

        @pl.when(c1 + 2 < chv)
        def _():
            start_gather(c1 + 2, buf1, sem1)

        pltpu.sync_copy(fbuf, accsh.at[col_v.at[c1]], add=True)
        return carry

    lax.fori_loop(0, chv // 2, body, 0)
    plsc.subcore_barrier()
    sl = pl.ds(s * ROWS_PER_SUB, ROWS_PER_SUB)
    pltpu.sync_copy(accsh.at[sl], out.at[c, sl])


def _make_edge_kernel(dw, dwb, ck):
    ch = EW0 // ck
    return functools.partial(
        pl.kernel,
        out_type=jax.ShapeDtypeStruct((NC, NP, dw), jnp.float32),
        mesh=_mesh,
        scratch_types=[
            pltpu.VMEM((ch, ck), jnp.int32),
            pltpu.VMEM((ch, ck), jnp.int32),
            pltpu.VMEM((EW0,), jnp.float32),
            pltpu.VMEM((ck, dwb), jnp.bfloat16),
            pltpu.VMEM((ck, dwb), jnp.bfloat16),
            pltpu.VMEM((ck, dw), jnp.float32),
            pltpu.SemaphoreType.DMA,
            pltpu.SemaphoreType.DMA,
            pltpu.VMEM_SHARED((NP, dw), jnp.float32),
        ],
        compiler_params=_sc_params,
    )(functools.partial(_edge_body, dw, dwb, ck))


CK_H = 64
CK_C = 128
C_BF = 64
_edge_kernel_h = _make_edge_kernel(H, H, CK_H)
_edge_kernel_c = _make_edge_kernel(C_PAD, C_BF, CK_C)


def _interleave_bf16(a):
    n, w = a.shape
    v = a.reshape(n, w // 32, 2, 16).swapaxes(2, 3).reshape(n, w)
    return v.astype(jnp.bfloat16)


_RB = 1024
_GRID = (NP // _RB,)


def _prep1_body(degp_ref, x_ref, w1i_ref, g1_ref, dis_ref):
    deg = degp_ref[0, :] + degp_ref[1, :]
    m = deg > 0.0
    dis = jnp.where(m, lax.rsqrt(jnp.where(m, deg, 1.0)), 0.0)
    h = jnp.dot(x_ref[...], w1i_ref[...], preferred_element_type=jnp.float32)
    g1_ref[...] = dis[:, None] * h
    dis_ref[...] = dis[:, None]


def _tc_prep1(degp, x, w1i):
    return pl.pallas_call(
        _prep1_body,
        grid=_GRID,
        in_specs=[
            pl.BlockSpec((NC, _RB), lambda i: (0, i)),
            pl.BlockSpec((_RB, D), lambda i: (i, 0)),
            pl.BlockSpec((D, H), lambda i: (0, 0)),
        ],
        out_specs=[
            pl.BlockSpec((_RB, H), lambda i: (i, 0)),
            pl.BlockSpec((_RB, 1), lambda i: (i, 0)),
        ],
        out_shape=[
            jax.ShapeDtypeStruct((N, H), jnp.float32),
            jax.ShapeDtypeStruct((N, 1), jnp.float32),
        ],
    )(degp, x, w1i)


def _mid_body(p_ref, dis_ref, x_ref, w1r_ref, b1_ref, w2i_ref, h_ref, g2_ref):
    agg = dis_ref[...] * (p_ref[0] + p_ref[1])
    root = jnp.dot(x_ref[...], w1r_ref[...], preferred_element_type=jnp.float32)
    h = jnp.maximum(agg + root + b1_ref[...], 0.0)
    h_ref[...] = h
    g2 = jnp.dot(h, w2i_ref[...], preferred_element_type=jnp.float32)
    g2_ref[...] = dis_ref[...] * g2


def _tc_mid(parts1, dis, x, w1r, b1, w2i):
    return pl.pallas_call(
        _mid_body,
        grid=_GRID,
        in_specs=[
            pl.BlockSpec((NC, _RB, H), lambda i: (0, i, 0)),
            pl.BlockSpec((_RB, 1), lambda i: (i, 0)),
            pl.BlockSpec((_RB, D), lambda i: (i, 0)),
            pl.BlockSpec((D, H), lambda i: (0, 0)),
            pl.BlockSpec((1, H), lambda i: (0, 0)),
            pl.BlockSpec((H, C_PAD), lambda i: (0, 0)),
        ],
        out_specs=[
            pl.BlockSpec((_RB, H), lambda i: (i, 0)),
            pl.BlockSpec((_RB, C_PAD), lambda i: (i, 0)),
        ],
        out_shape=[
            jax.ShapeDtypeStruct((N, H), jnp.float32),
            jax.ShapeDtypeStruct((N, C_PAD), jnp.float32),
        ],
    )(parts1, dis, x, w1r, b1, w2i)


def _post2_body(p_ref, dis_ref, h_ref, w2r_ref, b2_ref, out_ref):
    agg = dis_ref[...] * (p_ref[0, :, :C] + p_ref[1, :, :C])
    root = jnp.dot(h_ref[...], w2r_ref[...], preferred_element_type=jnp.float32)
    out_ref[...] = jnp.maximum(agg + root + b2_ref[...], 0.0)


def _tc_post2(parts2, dis, h, w2r, b2):
    return pl.pallas_call(
        _post2_body,
        grid=_GRID,
        in_specs=[
            pl.BlockSpec((NC, _RB, C_PAD), lambda i: (0, i, 0)),
            pl.BlockSpec((_RB, 1), lambda i: (i, 0)),
            pl.BlockSpec((_RB, H), lambda i: (i, 0)),
            pl.BlockSpec((H, C), lambda i: (0, 0)),
            pl.BlockSpec((1, C), lambda i: (0, 0)),
        ],
        out_specs=pl.BlockSpec((_RB, C), lambda i: (i, 0)),
        out_shape=jax.ShapeDtypeStruct((N, C), jnp.float32),
    )(parts2, dis, h, w2r, b2)


def kernel(x, edge_index, edge_weight, W1_init, W1_root, b1, W2_init,
           W2_root, b2):
    row = edge_index[0]
    col = edge_index[1]
    pad = EP - E
    colm = jnp.pad(col, (0, pad)).reshape(NW, CH, CK)
    wm = jnp.pad(edge_weight, (0, pad)).reshape(NW, CH, CK)

    def uneven(a):
        f = jnp.pad(a, (0, EP_U - E))
        p0 = f[: NS * EW0].reshape(NS, EW0)
        p1 = jnp.pad(f[NS * EW0:].reshape(NS, EW1), ((0, 0), (0, EW0 - EW1)))
        return jnp.concatenate([p0, p1], axis=0)

    rowu = uneven(row)
    colu = uneven(col)
    wu = uneven(edge_weight)

    zeros1 = jnp.zeros((ROWS_PER_SUB,), jnp.float32)
    zerosH = jnp.zeros((ROWS_PER_SUB, H), jnp.float32)
    zerosC = jnp.zeros((ROWS_PER_SUB, C_PAD), jnp.float32)
    w2i_p = jnp.pad(W2_init, ((0, 0), (0, C_PAD - C)))
    b1r = b1.reshape(1, H)
    b2r = b2.reshape(1, C)

    degp = _deg_kernel(colm, wm, zeros1)
    g1, dis = _tc_prep1(degp, x, W1_init)
    parts1 = _edge_kernel_h(
        _interleave_bf16(g1), rowu.reshape(NW, EW0 // CK_H, CK_H),
        colu.reshape(NW, EW0 // CK_H, CK_H), wu, zerosH)
    h, g2 = _tc_mid(parts1, dis, x, W1_root, b1r, w2i_p)
    g2p = jnp.pad(g2, ((0, 0), (0, C_BF - C_PAD)))
    parts2 = _edge_kernel_c(
        _interleave_bf16(g2p), rowu.reshape(NW, EW0 // CK_C, CK_C),
        colu.reshape(NW, EW0 // CK_C, CK_C), wu, zerosC)
    return _tc_post2(parts2, dis, h, W2_root, b2r)


# speedup vs baseline: 1.0516x; 1.0516x over previous
import functools

import jax
import jax.numpy as jnp
from jax import lax
from jax.experimental import pallas as pl
from jax.experimental.pallas import tpu as pltpu
from jax.experimental.pallas import tpu_sc as plsc

N = 10000
E = 320000
D = 128
H = 128
C = 40
C_PAD = 48

NC = 2
NS = 16
NW = NC * NS

NP = 10240
ROWS_PER_SUB = NP // NS

CK = 128
EW = 10240
CH = EW // CK
EP = EW * NW

EW0 = 10240
EW1 = 10240
EP_U = NS * (EW0 + EW1)

_mesh = plsc.VectorSubcoreMesh(core_axis_name="c", subcore_axis_name="s")
_sc_params = pltpu.CompilerParams(
    needs_layout_passes=False, use_tc_tiling_on_sc=False
)


def _full16(v):
    return jnp.full((16,), v, dtype=jnp.int32)


def _deg_body(colm, wm, zeros1, out, col_v, w_v, sem, degsh):
    c = lax.axis_index("c")
    s = lax.axis_index("s")
    wid = s * NC + c
    pltpu.sync_copy(colm.at[wid], col_v)
    pltpu.sync_copy(wm.at[wid], w_v)
    pltpu.sync_copy(zeros1, degsh.at[pl.ds(s * ROWS_PER_SUB, ROWS_PER_SUB)])
    plsc.subcore_barrier()

    def chunk(ci, carry):
        pltpu.sync_copy(w_v.at[ci], degsh.at[col_v.at[ci]], add=True)
        return carry

    lax.fori_loop(0, CH, chunk, 0)
    plsc.subcore_barrier()
    sl = pl.ds(s * ROWS_PER_SUB, ROWS_PER_SUB)
    pltpu.sync_copy(degsh.at[sl], out.at[c, sl])


_deg_kernel = functools.partial(
    pl.kernel,
    out_type=jax.ShapeDtypeStruct((NC, NP), jnp.float32),
    mesh=_mesh,
    scratch_types=[
        pltpu.VMEM((CH, CK), jnp.int32),
        pltpu.VMEM((CH, CK), jnp.float32),
        pltpu.SemaphoreType.DMA,
        pltpu.VMEM_SHARED((NP,), jnp.float32),
    ],
    compiler_params=_sc_params,
)(_deg_body)


def _edge_body(dw, dwb, ck, g, rowm, colm, wmf, zerosD, out, row_v, col_v,
               w_v, buf0, buf1, fbuf, sem0, sem1, accsh):
    ch0 = EW0 // ck
    ch1 = EW1 // ck
    c = lax.axis_index("c")
    s = lax.axis_index("s")
    wid = s * NC + c
    chv = ch0 if ch0 == ch1 else jnp.where(c == 0, ch0, ch1)
    pltpu.sync_copy(rowm.at[wid], row_v)
    pltpu.sync_copy(colm.at[wid], col_v)
    pltpu.sync_copy(wmf.at[wid], w_v)
    pltpu.sync_copy(zerosD, accsh.at[pl.ds(s * ROWS_PER_SUB, ROWS_PER_SUB)])
    plsc.subcore_barrier()

    mneg = jnp.full((16,), -65536, jnp.int32)

    def scale_rows(buf, ci):
        def block(b, bc):
            rb = b * 16
            for k in range(16):
                wspl = plsc.load_gather(w_v, [_full16(ci * ck + rb + k)])
                r = rb + k
                for j in range(dwb // 32):
                    v = plsc.bitcast(buf[r, pl.ds(j * 32, 32)], jnp.int32)
                    a = plsc.bitcast(v << 16, jnp.float32)
                    fbuf[r, pl.ds(j * 32, 16)] = a * wspl
                    if j * 32 + 16 < dw:
                        bb = plsc.bitcast(v & mneg, jnp.float32)
                        fbuf[r, pl.ds(j * 32 + 16, 16)] = bb * wspl
            return bc

        lax.fori_loop(0, ck // 16, block, 0)

    def start_gather(ci, buf, sem):
        pltpu.async_copy(g.at[row_v.at[ci]], buf, sem)

    def wait_gather(ci, buf, sem):
        pltpu.make_async_copy(g.at[row_v.at[ci]], buf, sem).wait()

    start_gather(0, buf0, sem0)
    start_gather(1, buf1, sem1)

    def body(k, carry):
        c0 = 2 * k
        c1 = c0 + 1
        wait_gather(c0, buf0, sem0)
        scale_rows(buf0, c0)

        @pl.when(c0 + 2 < chv)
        def _():
            start_gather(c0 + 2, buf0, sem0)

        pltpu.sync_copy(fbuf, accsh.at[col_v.at[c0]], add=True)

        wait_gather(c1, buf1, sem1)
        scale_rows(buf1, c1)

        @pl.when(c1 + 2 < chv)
        def _():
            start_gather(c1 + 2, buf1, sem1)

        pltpu.sync_copy(fbuf, accsh.at[col_v.at[c1]], add=True)
        return carry

    lax.fori_loop(0, chv // 2, body, 0)
    plsc.subcore_barrier()
    sl = pl.ds(s * ROWS_PER_SUB, ROWS_PER_SUB)
    pltpu.sync_copy(accsh.at[sl], out.at[c, sl])


def _make_edge_kernel(dw, dwb, ck):
    ch = EW0 // ck
    return functools.partial(
        pl.kernel,
        out_type=jax.ShapeDtypeStruct((NC, NP, dw), jnp.float32),
        mesh=_mesh,
        scratch_types=[
            pltpu.VMEM((ch, ck), jnp.int32),
            pltpu.VMEM((ch, ck), jnp.int32),
            pltpu.VMEM((EW0,), jnp.float32),
            pltpu.VMEM((ck, dwb), jnp.bfloat16),
            pltpu.VMEM((ck, dwb), jnp.bfloat16),
            pltpu.VMEM((ck, dw), jnp.float32),
            pltpu.SemaphoreType.DMA,
            pltpu.SemaphoreType.DMA,
            pltpu.VMEM_SHARED((NP, dw), jnp.float32),
        ],
        compiler_params=_sc_params,
    )(functools.partial(_edge_body, dw, dwb, ck))


CK_H = 64
CK_C = 128
C_BF = 64
_edge_kernel_h = _make_edge_kernel(H, H, CK_H)
_edge_kernel_c = _make_edge_kernel(C_PAD, C_BF, CK_C)


def _interleave_bf16(a):
    n, w = a.shape
    v = a.reshape(n, w // 32, 2, 16).swapaxes(2, 3).reshape(n, w)
    return v.astype(jnp.bfloat16)


_RB = 1024
_GRID = (NP // _RB,)


def _prep1_body(degp_ref, x_ref, w1i_ref, g1_ref, dis_ref):
    deg = degp_ref[0, :] + degp_ref[1, :]
    m = deg > 0.0
    dis = jnp.where(m, lax.rsqrt(jnp.where(m, deg, 1.0)), 0.0)
    h = jnp.dot(x_ref[...], w1i_ref[...], preferred_element_type=jnp.float32)
    g1_ref[...] = dis[:, None] * h
    dis_ref[...] = dis[:, None]


def _tc_prep1(degp, x, w1i):
    return pl.pallas_call(
        _prep1_body,
        grid=_GRID,
        in_specs=[
            pl.BlockSpec((NC, _RB), lambda i: (0, i)),
            pl.BlockSpec((_RB, D), lambda i: (i, 0)),
            pl.BlockSpec((D, H), lambda i: (0, 0)),
        ],
        out_specs=[
            pl.BlockSpec((_RB, H), lambda i: (i, 0)),
            pl.BlockSpec((_RB, 1), lambda i: (i, 0)),
        ],
        out_shape=[
            jax.ShapeDtypeStruct((N, H), jnp.float32),
            jax.ShapeDtypeStruct((N, 1), jnp.float32),
        ],
    )(degp, x, w1i)


def _mid_body(p_ref, dis_ref, x_ref, w1r_ref, b1_ref, w2i_ref, h_ref, g2_ref):
    agg = dis_ref[...] * (p_ref[0] + p_ref[1])
    root = jnp.dot(x_ref[...], w1r_ref[...], preferred_element_type=jnp.float32)
    h = jnp.maximum(agg + root + b1_ref[...], 0.0)
    h_ref[...] = h
    g2 = jnp.dot(h, w2i_ref[...], preferred_element_type=jnp.float32)
    g2_ref[...] = dis_ref[...] * g2


def _tc_mid(parts1, dis, x, w1r, b1, w2i):
    return pl.pallas_call(
        _mid_body,
        grid=_GRID,
        in_specs=[
            pl.BlockSpec((NC, _RB, H), lambda i: (0, i, 0)),
            pl.BlockSpec((_RB, 1), lambda i: (i, 0)),
            pl.BlockSpec((_RB, D), lambda i: (i, 0)),
            pl.BlockSpec((D, H), lambda i: (0, 0)),
            pl.BlockSpec((1, H), lambda i: (0, 0)),
            pl.BlockSpec((H, C_PAD), lambda i: (0, 0)),
        ],
        out_specs=[
            pl.BlockSpec((_RB, H), lambda i: (i, 0)),
            pl.BlockSpec((_RB, C_PAD), lambda i: (i, 0)),
        ],
        out_shape=[
            jax.ShapeDtypeStruct((N, H), jnp.float32),
            jax.ShapeDtypeStruct((N, C_PAD), jnp.float32),
        ],
    )(parts1, dis, x, w1r, b1, w2i)


def _post2_body(p_ref, dis_ref, h_ref, w2r_ref, b2_ref, out_ref):
    agg = dis_ref[...] * (p_ref[0, :, :C] + p_ref[1, :, :C])
    root = jnp.dot(h_ref[...], w2r_ref[...], preferred_element_type=jnp.float32)
    out_ref[...] = jnp.maximum(agg + root + b2_ref[...], 0.0)


def _tc_post2(parts2, dis, h, w2r, b2):
    return pl.pallas_call(
        _post2_body,
        grid=_GRID,
        in_specs=[
            pl.BlockSpec((NC, _RB, C_PAD), lambda i: (0, i, 0)),
            pl.BlockSpec((_RB, 1), lambda i: (i, 0)),
            pl.BlockSpec((_RB, H), lambda i: (i, 0)),
            pl.BlockSpec((H, C), lambda i: (0, 0)),
            pl.BlockSpec((1, C), lambda i: (0, 0)),
        ],
        out_specs=pl.BlockSpec((_RB, C), lambda i: (i, 0)),
        out_shape=jax.ShapeDtypeStruct((N, C), jnp.float32),
    )(parts2, dis, h, w2r, b2)


def kernel(x, edge_index, edge_weight, W1_init, W1_root, b1, W2_init,
           W2_root, b2):
    row = edge_index[0]
    col = edge_index[1]
    pad = EP - E
    rowm = jnp.pad(row, (0, pad)).reshape(NW, CH, CK)
    colm = jnp.pad(col, (0, pad)).reshape(NW, CH, CK)
    wm = jnp.pad(edge_weight, (0, pad)).reshape(NW, CH, CK)

    zeros1 = jnp.zeros((ROWS_PER_SUB,), jnp.float32)
    zerosH = jnp.zeros((ROWS_PER_SUB, H), jnp.float32)
    zerosC = jnp.zeros((ROWS_PER_SUB, C_PAD), jnp.float32)
    w2i_p = jnp.pad(W2_init, ((0, 0), (0, C_PAD - C)))
    b1r = b1.reshape(1, H)
    b2r = b2.reshape(1, C)

    wmf = wm.reshape(NW, EW)
    degp = _deg_kernel(colm, wm, zeros1)
    g1, dis = _tc_prep1(degp, x, W1_init)
    parts1 = _edge_kernel_h(
        _interleave_bf16(g1), rowm.reshape(NW, EW // CK_H, CK_H),
        colm.reshape(NW, EW // CK_H, CK_H), wmf, zerosH)
    h, g2 = _tc_mid(parts1, dis, x, W1_root, b1r, w2i_p)
    g2p = jnp.pad(g2, ((0, 0), (0, C_BF - C_PAD)))
    parts2 = _edge_kernel_c(
        _interleave_bf16(g2p), rowm.reshape(NW, EW // CK_C, CK_C),
        colm.reshape(NW, EW // CK_C, CK_C), wmf, zerosC)
    return _tc_post2(parts2, dis, h, W2_root, b2r)


# speedup vs baseline: 1.0855x; 1.0322x over previous
import functools

import jax
import jax.numpy as jnp
from jax import lax
from jax.experimental import pallas as pl
from jax.experimental.pallas import tpu as pltpu
from jax.experimental.pallas import tpu_sc as plsc

N = 10000
E = 320000
D = 128
H = 128
C = 40
C_PAD = 48

NC = 2
NS = 16
NW = NC * NS

NP = 10240
ROWS_PER_SUB = NP // NS

CK = 128
EW = 10240
CH = EW // CK
EP = EW * NW

EW0 = 10240
EW1 = 10240
EP_U = NS * (EW0 + EW1)

_mesh = plsc.VectorSubcoreMesh(core_axis_name="c", subcore_axis_name="s")
_sc_params = pltpu.CompilerParams(
    needs_layout_passes=False, use_tc_tiling_on_sc=False
)


def _full16(v):
    return jnp.full((16,), v, dtype=jnp.int32)


def _deg_body(colm, wm, zeros1, out, col_v, w_v, sem, degsh):
    c = lax.axis_index("c")
    s = lax.axis_index("s")
    wid = s * NC + c
    pltpu.sync_copy(colm.at[wid], col_v)
    pltpu.sync_copy(wm.at[wid], w_v)
    pltpu.sync_copy(zeros1, degsh.at[pl.ds(s * ROWS_PER_SUB, ROWS_PER_SUB)])
    plsc.subcore_barrier()

    def chunk(ci, carry):
        pltpu.sync_copy(w_v.at[ci], degsh.at[col_v.at[ci]], add=True)
        return carry

    lax.fori_loop(0, CH, chunk, 0)
    plsc.subcore_barrier()
    sl = pl.ds(s * ROWS_PER_SUB, ROWS_PER_SUB)
    pltpu.sync_copy(degsh.at[sl], out.at[c, sl])


_deg_kernel = functools.partial(
    pl.kernel,
    out_type=jax.ShapeDtypeStruct((NC, NP), jnp.float32),
    mesh=_mesh,
    scratch_types=[
        pltpu.VMEM((CH, CK), jnp.int32),
        pltpu.VMEM((CH, CK), jnp.float32),
        pltpu.SemaphoreType.DMA,
        pltpu.VMEM_SHARED((NP,), jnp.float32),
    ],
    compiler_params=_sc_params,
)(_deg_body)


def _edge_body(dw, dwb, ck, g, rowm, colm, wmf, zerosD, out, row_v, col_v,
               w_v, buf0, buf1, fbuf, sem0, sem1, accsh):
    ch0 = EW0 // ck
    ch1 = EW1 // ck
    c = lax.axis_index("c")
    s = lax.axis_index("s")
    wid = s * NC + c
    chv = ch0 if ch0 == ch1 else jnp.where(c == 0, ch0, ch1)
    pltpu.sync_copy(rowm.at[wid], row_v)
    pltpu.sync_copy(colm.at[wid], col_v)
    pltpu.sync_copy(wmf.at[wid], w_v)
    pltpu.sync_copy(zerosD, accsh.at[pl.ds(s * ROWS_PER_SUB, ROWS_PER_SUB)])
    plsc.subcore_barrier()

    mneg = jnp.full((16,), -65536, jnp.int32)

    def scale_rows(buf, ci):
        def block(b, bc):
            rb = b * 16
            for k in range(16):
                wspl = plsc.load_gather(w_v, [_full16(ci * ck + rb + k)])
                r = rb + k
                for j in range(dwb // 32):
                    v = plsc.bitcast(buf[r, pl.ds(j * 32, 32)], jnp.int32)
                    a = plsc.bitcast(v << 16, jnp.float32)
                    fbuf[r, pl.ds(j * 32, 16)] = a * wspl
                    if j * 32 + 16 < dw:
                        bb = plsc.bitcast(v & mneg, jnp.float32)
                        fbuf[r, pl.ds(j * 32 + 16, 16)] = bb * wspl
            return bc

        lax.fori_loop(0, ck // 16, block, 0)

    def start_gather(ci, buf, sem):
        pltpu.async_copy(g.at[row_v.at[ci]], buf, sem)

    def wait_gather(ci, buf, sem):
        pltpu.make_async_copy(g.at[row_v.at[ci]], buf, sem).wait()

    start_gather(0, buf0, sem0)
    start_gather(1, buf1, sem1)

    def body(k, carry):
        c0 = 2 * k
        c1 = c0 + 1
        wait_gather(c0, buf0, sem0)
        scale_rows(buf0, c0)

        @pl.when(c0 + 2 < chv)
        def _():
            start_gather(c0 + 2, buf0, sem0)

        pltpu.sync_copy(fbuf, accsh.at[col_v.at[c0]], add=True)

        wait_gather(c1, buf1, sem1)
        scale_rows(buf1, c1)

        @pl.when(c1 + 2 < chv)
        def _():
            start_gather(c1 + 2, buf1, sem1)

        pltpu.sync_copy(fbuf, accsh.at[col_v.at[c1]], add=True)
        return carry

    lax.fori_loop(0, chv // 2, body, 0)
    plsc.subcore_barrier()
    sl = pl.ds(s * ROWS_PER_SUB, ROWS_PER_SUB)
    pltpu.sync_copy(accsh.at[sl], out.at[c, sl])


def _make_edge_kernel(dw, dwb, ck):
    ch = EW0 // ck
    return functools.partial(
        pl.kernel,
        out_type=jax.ShapeDtypeStruct((NC, NP, dw), jnp.float32),
        mesh=_mesh,
        scratch_types=[
            pltpu.VMEM((ch, ck), jnp.int32),
            pltpu.VMEM((ch, ck), jnp.int32),
            pltpu.VMEM((EW0,), jnp.float32),
            pltpu.VMEM((ck, dwb), jnp.bfloat16),
            pltpu.VMEM((ck, dwb), jnp.bfloat16),
            pltpu.VMEM((ck, dw), jnp.float32),
            pltpu.SemaphoreType.DMA,
            pltpu.SemaphoreType.DMA,
            pltpu.VMEM_SHARED((NP, dw), jnp.float32),
        ],
        compiler_params=_sc_params,
    )(functools.partial(_edge_body, dw, dwb, ck))


CK_H = 64
CK_C = 128
C_BF = 64
_edge_kernel_h = _make_edge_kernel(H, H, CK_H)
_edge_kernel_c = _make_edge_kernel(C_PAD, C_BF, CK_C)


def _perm(w):
    p = []
    for g in range(w // 32):
        for i in range(16):
            p.extend((32 * g + i, 32 * g + 16 + i))
    return jnp.array(p, dtype=jnp.int32)


_RB = 1024
_GRID = (NP // _RB,)


def _prep1_body(degp_ref, x_ref, w1i_ref, g1_ref, dis_ref):
    deg = degp_ref[0, :] + degp_ref[1, :]
    m = deg > 0.0
    dis = jnp.where(m, lax.rsqrt(jnp.where(m, deg, 1.0)), 0.0)
    h = jnp.dot(x_ref[...], w1i_ref[...], preferred_element_type=jnp.float32)
    g1_ref[...] = (dis[:, None] * h).astype(jnp.bfloat16)
    dis_ref[...] = dis[:, None]


def _tc_prep1(degp, x, w1i):
    return pl.pallas_call(
        _prep1_body,
        grid=_GRID,
        in_specs=[
            pl.BlockSpec((NC, _RB), lambda i: (0, i)),
            pl.BlockSpec((_RB, D), lambda i: (i, 0)),
            pl.BlockSpec((D, H), lambda i: (0, 0)),
        ],
        out_specs=[
            pl.BlockSpec((_RB, H), lambda i: (i, 0)),
            pl.BlockSpec((_RB, 1), lambda i: (i, 0)),
        ],
        out_shape=[
            jax.ShapeDtypeStruct((N, H), jnp.bfloat16),
            jax.ShapeDtypeStruct((N, 1), jnp.float32),
        ],
    )(degp, x, w1i)


def _mid_body(p_ref, dis_ref, x_ref, w1r_ref, b1_ref, w2i_ref, h_ref, g2_ref):
    agg = dis_ref[...] * (p_ref[0] + p_ref[1])
    root = jnp.dot(x_ref[...], w1r_ref[...], preferred_element_type=jnp.float32)
    h = jnp.maximum(agg + root + b1_ref[...], 0.0)
    h_ref[...] = h
    g2 = jnp.dot(h, w2i_ref[...], preferred_element_type=jnp.float32)
    g2_ref[...] = (dis_ref[...] * g2).astype(jnp.bfloat16)


def _tc_mid(parts1, dis, x, w1r, b1, w2i):
    return pl.pallas_call(
        _mid_body,
        grid=_GRID,
        in_specs=[
            pl.BlockSpec((NC, _RB, H), lambda i: (0, i, 0)),
            pl.BlockSpec((_RB, 1), lambda i: (i, 0)),
            pl.BlockSpec((_RB, D), lambda i: (i, 0)),
            pl.BlockSpec((D, H), lambda i: (0, 0)),
            pl.BlockSpec((1, H), lambda i: (0, 0)),
            pl.BlockSpec((H, C_BF), lambda i: (0, 0)),
        ],
        out_specs=[
            pl.BlockSpec((_RB, H), lambda i: (i, 0)),
            pl.BlockSpec((_RB, C_BF), lambda i: (i, 0)),
        ],
        out_shape=[
            jax.ShapeDtypeStruct((N, H), jnp.float32),
            jax.ShapeDtypeStruct((N, C_BF), jnp.bfloat16),
        ],
    )(parts1, dis, x, w1r, b1, w2i)


def _post2_body(p_ref, dis_ref, h_ref, w2r_ref, b2_ref, out_ref):
    agg = dis_ref[...] * (p_ref[0, :, :C] + p_ref[1, :, :C])
    root = jnp.dot(h_ref[...], w2r_ref[...], preferred_element_type=jnp.float32)
    out_ref[...] = jnp.maximum(agg + root + b2_ref[...], 0.0)


def _tc_post2(parts2, dis, h, w2r, b2):
    return pl.pallas_call(
        _post2_body,
        grid=_GRID,
        in_specs=[
            pl.BlockSpec((NC, _RB, C_PAD), lambda i: (0, i, 0)),
            pl.BlockSpec((_RB, 1), lambda i: (i, 0)),
            pl.BlockSpec((_RB, H), lambda i: (i, 0)),
            pl.BlockSpec((H, C), lambda i: (0, 0)),
            pl.BlockSpec((1, C), lambda i: (0, 0)),
        ],
        out_specs=pl.BlockSpec((_RB, C), lambda i: (i, 0)),
        out_shape=jax.ShapeDtypeStruct((N, C), jnp.float32),
    )(parts2, dis, h, w2r, b2)


def kernel(x, edge_index, edge_weight, W1_init, W1_root, b1, W2_init,
           W2_root, b2):
    row = edge_index[0]
    col = edge_index[1]
    pad = EP - E
    rowm = jnp.pad(row, (0, pad)).reshape(NW, CH, CK)
    colm = jnp.pad(col, (0, pad)).reshape(NW, CH, CK)
    wm = jnp.pad(edge_weight, (0, pad)).reshape(NW, CH, CK)

    zeros1 = jnp.zeros((ROWS_PER_SUB,), jnp.float32)
    zerosH = jnp.zeros((ROWS_PER_SUB, H), jnp.float32)
    zerosC = jnp.zeros((ROWS_PER_SUB, C_PAD), jnp.float32)
    w1i_s = W1_init[:, _perm(H)]
    w2i_s = jnp.pad(W2_init, ((0, 0), (0, C_BF - C)))[:, _perm(C_BF)]
    b1r = b1.reshape(1, H)
    b2r = b2.reshape(1, C)

    wmf = wm.reshape(NW, EW)
    degp = _deg_kernel(colm, wm, zeros1)
    g1, dis = _tc_prep1(degp, x, w1i_s)
    parts1 = _edge_kernel_h(
        g1, rowm.reshape(NW, EW // CK_H, CK_H),
        colm.reshape(NW, EW // CK_H, CK_H), wmf, zerosH)
    h, g2 = _tc_mid(parts1, dis, x, W1_root, b1r, w2i_s)
    parts2 = _edge_kernel_c(
        g2, rowm.reshape(NW, EW // CK_C, CK_C),
        colm.reshape(NW, EW // CK_C, CK_C), wmf, zerosC)
    return _tc_post2(parts2, dis, h, W2_root, b2r)
